# Initial kernel scaffold; baseline (speedup 1.0000x reference)
import jax
import jax.numpy as jnp
from jax.experimental import pallas as pl


def kernel(x, Wg, bg, Wn, bn, W1, b1, W2, b2):
    raise NotImplementedError("write your pallas kernel here")


# speedup vs baseline: 1.0212x; 1.0212x over previous
import functools

import jax
import jax.numpy as jnp
from jax.experimental import pallas as pl
from jax.experimental.pallas import tpu as pltpu


def _router_body(x_ref, wg_ref, bg_ref, wn_ref, bn_ref, noise_ref, gate_ref):
    x = x_ref[...]
    logits = jnp.dot(x, wg_ref[...], preferred_element_type=jnp.float32) + bg_ref[...]
    nlog = jnp.dot(x, wn_ref[...], preferred_element_type=jnp.float32) + bn_ref[...]
    sp = jnp.maximum(nlog, 0.0) + jnp.log1p(jnp.exp(-jnp.abs(nlog)))
    noisy = logits + noise_ref[...] * sp

    E = noisy.shape[-1]
    col = jax.lax.broadcasted_iota(jnp.int32, noisy.shape, 1)
    m1 = jnp.max(noisy, axis=1, keepdims=True)
    i1 = jnp.min(jnp.where(noisy == m1, col, E), axis=1, keepdims=True)
    masked = jnp.where(col == i1, -jnp.inf, noisy)
    m2 = jnp.max(masked, axis=1, keepdims=True)
    i2 = jnp.min(jnp.where(masked == m2, col, E), axis=1, keepdims=True)
    sel = (col == i1) | (col == i2)
    w = jnp.where(sel, jnp.exp(noisy - m1), 0.0)
    gate_ref[...] = w / jnp.sum(w, axis=1, keepdims=True)


def _ffn_body(gate_ref, x_ref, w1_ref, b1_ref, w2_ref, b2_ref, out_ref,
              acc_ref, *, tb, n_exp):
    e = pl.program_id(0)
    t = pl.program_id(1)

    xb = x_ref[...]
    h = jnp.dot(xb, w1_ref[0], preferred_element_type=jnp.float32) + b1_ref[0, 0]
    h = jnp.maximum(h, 0.0).astype(jnp.bfloat16)
    eo = jnp.dot(h, w2_ref[0], preferred_element_type=jnp.float32) + b2_ref[0, 0]

    gate = gate_ref[...]
    col = jax.lax.broadcasted_iota(jnp.int32, gate.shape, 1)
    g_e = jnp.sum(jnp.where(col == e, gate, 0.0), axis=1, keepdims=True)

    rows = pl.ds(t * tb, tb)
    prev = jnp.where(e == 0, 0.0, acc_ref[rows, :])
    new = prev + g_e * eo
    acc_ref[rows, :] = new

    @pl.when(e == n_exp - 1)
    def _():
        out_ref[...] = new


def kernel(x, Wg, bg, Wn, bn, W1, b1, W2, b2):
    N, D = x.shape
    E = Wg.shape[1]
    H = W1.shape[2]

    noise = jax.random.normal(jax.random.key(42), (N, E), jnp.float32)

    tb_r = 256
    gating = pl.pallas_call(
        _router_body,
        grid=(N // tb_r,),
        in_specs=[
            pl.BlockSpec((tb_r, D), lambda t: (t, 0)),
            pl.BlockSpec((D, E), lambda t: (0, 0)),
            pl.BlockSpec((1, E), lambda t: (0, 0)),
            pl.BlockSpec((D, E), lambda t: (0, 0)),
            pl.BlockSpec((1, E), lambda t: (0, 0)),
            pl.BlockSpec((tb_r, E), lambda t: (t, 0)),
        ],
        out_specs=pl.BlockSpec((tb_r, E), lambda t: (t, 0)),
        out_shape=jax.ShapeDtypeStruct((N, E), jnp.float32),
    )(x, Wg, bg.reshape(1, E), Wn, bn.reshape(1, E), noise)

    xb = x.astype(jnp.bfloat16)
    W1b = W1.astype(jnp.bfloat16)
    W2b = W2.astype(jnp.bfloat16)

    tb = 256
    out = pl.pallas_call(
        functools.partial(_ffn_body, tb=tb, n_exp=E),
        grid=(E, N // tb),
        in_specs=[
            pl.BlockSpec((tb, E), lambda e, t: (t, 0)),
            pl.BlockSpec((tb, D), lambda e, t: (t, 0)),
            pl.BlockSpec((1, D, H), lambda e, t: (e, 0, 0)),
            pl.BlockSpec((1, 1, H), lambda e, t: (e, 0, 0)),
            pl.BlockSpec((1, H, D), lambda e, t: (e, 0, 0)),
            pl.BlockSpec((1, 1, D), lambda e, t: (e, 0, 0)),
        ],
        out_specs=pl.BlockSpec((tb, D), lambda e, t: (t, 0)),
        out_shape=jax.ShapeDtypeStruct((N, D), jnp.float32),
        scratch_shapes=[pltpu.VMEM((N, D), jnp.float32)],
    )(gating, xb, W1b, b1.reshape(E, 1, H), W2b, b2.reshape(E, 1, D))
    return out
